# baseline (speedup 1.0000x reference)
import jax
import jax.numpy as jnp
from jax import lax
from jax.experimental import pallas as pl

N = 4096
THRESHOLD = 1.0
ALPHA_PLUS = 0.01
INV_2TAU2 = 1.0 / (2.0 * 20.0 * 20.0)
BLOCK_R = 512
BAND_W = BLOCK_R + 128


def _make_block(base_block):
    def _gstdp_block(spikes_ref, row_spikes_ref, w_ref, spikes_out_ref, w_out_ref):
        r = pl.program_id(0) + base_block
        s = spikes_ref[...]
        mask = (s >= THRESHOLD).astype(jnp.float32)
        spikes_out_ref[...] = mask
        many = jnp.sum(mask) > 1.0

        @pl.when(jnp.logical_not(many))
        def _passthrough():
            w_out_ref[...] = w_ref[...]

        @pl.when(many)
        def _update():
            w = w_ref[...]
            w_out_ref[...] = jnp.clip(w, 0.0, 1.0)
            start = jnp.minimum(r * BLOCK_R, N - BAND_W)
            row_mask = (row_spikes_ref[...] >= THRESHOLD).astype(jnp.float32)
            col_mask = (spikes_ref[pl.ds(start, BAND_W)] >= THRESHOLD).astype(
                jnp.float32
            )
            wb = w_ref[:, pl.ds(start, BAND_W)]
            col = lax.broadcasted_iota(jnp.int32, (BLOCK_R, BAND_W), 1) + start
            row = (
                lax.broadcasted_iota(jnp.int32, (BLOCK_R, BAND_W), 0) + r * BLOCK_R
            )
            d = (col - row).astype(jnp.float32)
            ltp = ALPHA_PLUS * jnp.exp(-(d * d) * INV_2TAU2)
            term = ltp * row_mask[:, None] * col_mask[None, :]
            term = jnp.where(col > row, term, 0.0)
            w_out_ref[:, pl.ds(start, BAND_W)] = jnp.clip(wb + term, 0.0, 1.0)

    return _gstdp_block


def _run_rows(input_spikes, weights, base_block, nblocks):
    return pl.pallas_call(
        _make_block(base_block),
        grid=(nblocks,),
        in_specs=[
            pl.BlockSpec((N,), lambda i: (0,)),
            pl.BlockSpec((BLOCK_R,), lambda i, b=base_block: (i + b,)),
            pl.BlockSpec((BLOCK_R, N), lambda i, b=base_block: (i + b, 0)),
        ],
        out_specs=[
            pl.BlockSpec((N,), lambda i: (0,)),
            pl.BlockSpec((BLOCK_R, N), lambda i: (i, 0)),
        ],
        out_shape=[
            jax.ShapeDtypeStruct((N,), jnp.float32),
            jax.ShapeDtypeStruct((nblocks * BLOCK_R, N), jnp.float32),
        ],
    )(input_spikes, input_spikes, weights)


@jax.jit
def kernel(input_spikes, weights):
    spikes, w0 = _run_rows(input_spikes, weights, 0, 4)
    _, w1 = _run_rows(input_spikes, weights, 4, 4)
    return spikes, jnp.concatenate([w0, w1], axis=0)
